# baseline (speedup 1.0000x reference)
import functools

import jax
import jax.numpy as jnp
from jax import lax
from jax.experimental import pallas as pl
from jax.experimental.pallas import tpu as pltpu
from jax.experimental.pallas import tpu_sc as plsc

N = 10000
E = 320000
D = 128
HID = 128
H = 8
C = 16
NODE_CLS = 8
EDGE_CLS = 6

NC = 2
NS = 16
NW = NC * NS
EPW = E // NW
GCH = 400
GSUB = 80
NPAD = 10240
ACCW = 144
CPT = ACCW // NS
SCH = 640
EPC = E // NC

_mesh = plsc.VectorSubcoreMesh(
    core_axis_name="c", subcore_axis_name="s", num_cores=NC, num_subcores=NS)


def _full_spec(shape):
    nd = len(shape)
    return pl.BlockSpec(shape, lambda *_i, _nd=nd: (0,) * _nd)


def _mm_bias(a, w, b, blk):
    r, k0 = a.shape
    k = w.shape[1]

    def body(a_r, w_r, b_r, o_r):
        o_r[...] = (
            jnp.dot(a_r[...], w_r[...], preferred_element_type=jnp.float32)
            + b_r[...])

    return pl.pallas_call(
        body,
        grid=(r // blk,),
        in_specs=[
            pl.BlockSpec((blk, k0), lambda i: (i, 0)),
            _full_spec(w.shape),
            _full_spec(b.shape),
        ],
        out_specs=pl.BlockSpec((blk, k), lambda i: (i, 0)),
        out_shape=jax.ShapeDtypeStruct((r, k), jnp.float32),
    )(a, w, b)


def _fold_weights(wup, bup, we1, we2):
    def body(wu, bu, w1, w2, m1, m2, c1, c2):
        wuv = wu[...]
        buv = bu[...]
        m1[...] = jnp.dot(wuv, w1[...], preferred_element_type=jnp.float32)
        m2[...] = jnp.dot(wuv, w2[...], preferred_element_type=jnp.float32)
        c1[...] = jnp.dot(buv, w1[...], preferred_element_type=jnp.float32)
        c2[...] = jnp.dot(buv, w2[...], preferred_element_type=jnp.float32)

    return pl.pallas_call(
        body,
        in_specs=[_full_spec(wup.shape), _full_spec(bup.shape),
                  _full_spec(we1.shape), _full_spec(we2.shape)],
        out_specs=[_full_spec((HID, HID)), _full_spec((HID, HID)),
                   _full_spec((1, HID)), _full_spec((1, HID))],
        out_shape=[jax.ShapeDtypeStruct((HID, HID), jnp.float32),
                   jax.ShapeDtypeStruct((HID, HID), jnp.float32),
                   jax.ShapeDtypeStruct((1, HID), jnp.float32),
                   jax.ShapeDtypeStruct((1, HID), jnp.float32)],
    )(wup, bup, we1, we2)


def _gat_edge(ef, g1, g2, m_w, cb, att_a, blk=2560):
    def body(ef_r, g1_r, g2_r, mw_r, cb_r, a_r, o_r):
        g1v = g1_r[...]
        eap = (jnp.dot(ef_r[...], mw_r[...], preferred_element_type=jnp.float32)
               + cb_r[...])
        m = g1v + g2_r[...] + eap
        m = jnp.maximum(m, 0.2 * m)
        logits = jnp.dot(m, a_r[...], preferred_element_type=jnp.float32)
        e = jnp.exp(jnp.minimum(logits, 75.0))
        parts = [g1v[:, 16 * h:16 * (h + 1)] * e[:, h:h + 1] for h in range(H)]
        parts.append(e)
        parts.append(jnp.zeros((blk, ACCW - 136), jnp.float32))
        o_r[...] = jnp.concatenate(parts, axis=1)

    return pl.pallas_call(
        body,
        grid=(E // blk,),
        in_specs=[
            pl.BlockSpec((blk, D), lambda i: (i, 0)),
            pl.BlockSpec((blk, HID), lambda i: (i, 0)),
            pl.BlockSpec((blk, HID), lambda i: (i, 0)),
            _full_spec(m_w.shape),
            _full_spec(cb.shape),
            _full_spec(att_a.shape),
        ],
        out_specs=pl.BlockSpec((blk, ACCW), lambda i: (i, 0)),
        out_shape=jax.ShapeDtypeStruct((E, ACCW), jnp.float32),
    )(ef, g1, g2, m_w, cb, att_a)


def _finalize(a0, a1, bias, ln_g, ln_b, blk=2000):
    def body(a0_r, a1_r, b_r, g_r, bb_r, o_r):
        t = a0_r[...] + a1_r[...]
        cols = [t[:, 16 * h:16 * (h + 1)] / (t[:, 128 + h:129 + h] + 1e-16)
                for h in range(H)]
        o = jnp.concatenate(cols, axis=1) + b_r[...]
        mu = jnp.mean(o, axis=1, keepdims=True)
        var = jnp.mean((o - mu) * (o - mu), axis=1, keepdims=True)
        o = (o - mu) * lax.rsqrt(var + 1e-5) * g_r[...] + bb_r[...]
        o_r[...] = jnp.maximum(o, 0.0)

    return pl.pallas_call(
        body,
        grid=(N // blk,),
        in_specs=[
            pl.BlockSpec((blk, ACCW), lambda i: (i, 0)),
            pl.BlockSpec((blk, ACCW), lambda i: (i, 0)),
            _full_spec(bias.shape),
            _full_spec(ln_g.shape),
            _full_spec(ln_b.shape),
        ],
        out_specs=pl.BlockSpec((blk, HID), lambda i: (i, 0)),
        out_shape=jax.ShapeDtypeStruct((N, HID), jnp.float32),
    )(a0, a1, bias, ln_g, ln_b)


def _node_head(h, w_all, b_all, blk=2000):
    k = w_all.shape[1]

    def body(h_r, w_r, b_r, o_np, o_rec, o_v, o_uz):
        z = (jnp.dot(h_r[...], w_r[...], preferred_element_type=jnp.float32)
             + b_r[...])
        zn = z[:, :8]
        zn = zn - jnp.max(zn, axis=1, keepdims=True)
        ez = jnp.exp(zn)
        o_np[...] = ez / jnp.sum(ez, axis=1, keepdims=True)
        o_rec[...] = z[:, 8:136]
        o_v[...] = z[:, 136:264]
        o_uz[...] = z[:, 264:520]

    return pl.pallas_call(
        body,
        grid=(N // blk,),
        in_specs=[
            pl.BlockSpec((blk, HID), lambda i: (i, 0)),
            _full_spec(w_all.shape),
            _full_spec(b_all.shape),
        ],
        out_specs=[
            pl.BlockSpec((blk, NODE_CLS), lambda i: (i, 0)),
            pl.BlockSpec((blk, D), lambda i: (i, 0)),
            pl.BlockSpec((blk, HID), lambda i: (i, 0)),
            pl.BlockSpec((blk, 256), lambda i: (i, 0)),
        ],
        out_shape=[
            jax.ShapeDtypeStruct((N, NODE_CLS), jnp.float32),
            jax.ShapeDtypeStruct((N, D), jnp.float32),
            jax.ShapeDtypeStruct((N, HID), jnp.float32),
            jax.ShapeDtypeStruct((N, 256), jnp.float32),
        ],
    )(h, w_all, b_all)


def _head_edge(guz, gv, w2row, b2, blk=2000):
    def body(guz_r, gv_r, w2_r, b2_r, o_ep, o_ex):
        guz = guz_r[...]
        zz = guz[:, 128:128 + EDGE_CLS]
        zz = zz - jnp.max(zz, axis=1, keepdims=True)
        ez = jnp.exp(zz)
        o_ep[...] = ez / jnp.sum(ez, axis=1, keepdims=True)
        hid = jnp.maximum(guz[:, :128] + gv_r[...], 0.0)
        s = jnp.sum(hid * w2_r[...], axis=1, keepdims=True) + b2_r[...]
        o_ex[...] = 1.0 / (1.0 + jnp.exp(-s))

    return pl.pallas_call(
        body,
        grid=(E // blk,),
        in_specs=[
            pl.BlockSpec((blk, 256), lambda i: (i, 0)),
            pl.BlockSpec((blk, HID), lambda i: (i, 0)),
            _full_spec(w2row.shape),
            _full_spec(b2.shape),
        ],
        out_specs=[
            pl.BlockSpec((blk, EDGE_CLS), lambda i: (i, 0)),
            pl.BlockSpec((blk, 1), lambda i: (i, 0)),
        ],
        out_shape=[
            jax.ShapeDtypeStruct((E, EDGE_CLS), jnp.float32),
            jax.ShapeDtypeStruct((E, 1), jnp.float32),
        ],
    )(guz, gv, w2row, b2)


def _sc_gather(tables, idxs, gch=80):
    nt = len(tables)
    ks = [int(t.shape[1]) for t in tables]
    out_type = tuple(
        jax.ShapeDtypeStruct((E, k), jnp.float32) for k in ks)
    scratch = []
    for k in ks:
        scratch += [
            pltpu.VMEM((gch,), jnp.int32),
            pltpu.VMEM((gch,), jnp.int32),
            pltpu.VMEM((gch, k), jnp.float32),
            pltpu.VMEM((gch, k), jnp.float32),
            pltpu.SemaphoreType.DMA,
            pltpu.SemaphoreType.DMA,
        ]
    nch = EPW // gch

    def body(*refs):
        tabs = refs[:nt]
        idxr = refs[nt:2 * nt]
        outs = refs[2 * nt:3 * nt]
        scr = refs[3 * nt:]
        ivs = [(scr[6 * t], scr[6 * t + 1]) for t in range(nt)]
        rvs = [(scr[6 * t + 2], scr[6 * t + 3]) for t in range(nt)]
        sems = [(scr[6 * t + 4], scr[6 * t + 5]) for t in range(nt)]
        cid = lax.axis_index("c")
        sid = lax.axis_index("s")
        wid = sid * NC + cid

        def base(i):
            return pl.multiple_of(wid * EPW + i * gch, gch)

        def stage(i, b):
            for t in range(nt):
                pltpu.sync_copy(idxr[t].at[pl.ds(base(i), gch)], ivs[t][b])
                pltpu.async_copy(tabs[t].at[ivs[t][b]], rvs[t][b],
                                 sems[t][b])

        def drain(i, b):
            for t in range(nt):
                pltpu.make_async_copy(tabs[t].at[ivs[t][b]], rvs[t][b],
                                      sems[t][b]).wait()
                pltpu.sync_copy(rvs[t][b], outs[t].at[pl.ds(base(i), gch)])

        stage(0, 0)

        def pairstep(ii, carry):
            i1 = 2 * ii + 1
            stage(i1, 1)
            drain(i1 - 1, 0)
            stage(i1 + 1, 0)
            drain(i1, 1)
            return carry

        lax.fori_loop(0, (nch - 1) // 2, pairstep, 0)
        drain(nch - 1, 0)

    fn = pl.kernel(body, out_type=out_type, mesh=_mesh,
                   scratch_types=tuple(scratch))
    return fn(*tables, *idxs)


def _sc_scatter_add(rows_t, dst):
    out_type = jax.ShapeDtypeStruct((NC, NS, NPAD * CPT), jnp.float32)
    scratch = [
        pltpu.VMEM((CPT, SCH), jnp.float32),
        pltpu.VMEM((CPT, SCH), jnp.float32),
        pltpu.VMEM((SCH,), jnp.int32),
        pltpu.VMEM((SCH,), jnp.int32),
        pltpu.VMEM((NPAD * CPT,), jnp.float32),
        pltpu.SemaphoreType.DMA,
        pltpu.SemaphoreType.DMA,
    ]
    nch = EPC // SCH

    def body(rows_r, dst_r, out_r, rv0, rv1, dv0, dv1, acc, sem0, sem1):
        cid = lax.axis_index("c")
        sid = lax.axis_index("s")
        rvs, dvs, sems = (rv0, rv1), (dv0, dv1), (sem0, sem1)

        def start(i, b):
            eb = pl.multiple_of(cid * EPC + i * SCH, SCH)
            pltpu.async_copy(rows_r.at[sid, :, pl.ds(eb, SCH)], rvs[b],
                             sems[b])
            pltpu.async_copy(dst_r.at[pl.ds(eb, SCH)], dvs[b], sems[b])

        def wait(b):
            pltpu.make_async_copy(rows_r.at[sid, :, pl.ds(0, SCH)], rvs[b],
                                  sems[b]).wait()
            pltpu.make_async_copy(dst_r.at[pl.ds(0, SCH)], dvs[b],
                                  sems[b]).wait()

        def compute(b):
            rv, dv = rvs[b], dvs[b]

            def group(g, carry2):
                dstv = dv[pl.ds(g * 16, 16)]
                lane = lax.iota(jnp.int32, 16)
                base_idx = dstv * CPT
                for k in range(CPT):
                    plsc.addupdate_scatter(acc, [base_idx + k],
                                           rv[k, pl.ds(g * 16, 16)],
                                           mask=lane >= 0)
                return carry2

            lax.fori_loop(0, SCH // 16, group, 0)

        def zstep(i, carry):
            acc[pl.ds(i * 16, 16)] = jnp.zeros((16,), jnp.float32)
            return carry

        start(0, 0)
        lax.fori_loop(0, NPAD * CPT // 16, zstep, 0)

        def step(ii, carry):
            i0 = 2 * ii
            start(i0 + 1, 1)
            wait(0)
            compute(0)
            start(lax.rem(i0 + 2, nch), 0)
            wait(1)
            compute(1)
            return carry

        lax.fori_loop(0, nch // 2, step, 0)
        wait(0)
        pltpu.sync_copy(acc, out_r.at[cid, sid])

    fn = pl.kernel(
        body, out_type=out_type, mesh=_mesh, scratch_types=scratch,
        compiler_params=pltpu.CompilerParams(needs_layout_passes=False))
    return fn(rows_t, dst)


def _att_matrix(att):
    eye = jnp.eye(H, dtype=jnp.float32)
    return (att[:, :, None] * eye[:, None, :]).reshape(H * C, H)


def kernel(x, edge_features, edge_index, params):
    src = edge_index[0]
    dst = edge_index[1]

    up = params["up"]
    wup = up["W"]
    bup = up["b"].reshape(1, HID)
    blk0, blk1 = params["blocks"]

    m1, m2, cb1, cb2 = _fold_weights(wup, bup, blk0["We"], blk1["We"])

    h = _mm_bias(x, wup, bup, 2000)

    for bp, m_w, cb in ((blk0, m1, cb1), (blk1, m2, cb2)):
        xl = _mm_bias(h, bp["Wl"], bp["bl"].reshape(1, H * C), 2000)
        xr = _mm_bias(h, bp["Wr"], bp["br"].reshape(1, H * C), 2000)
        g1, g2 = _sc_gather([xl, xr], [src, dst])
        wrows_t = _gat_edge(edge_features, g1, g2, m_w, cb,
                            _att_matrix(bp["att"]))
        acc3 = _sc_scatter_add(wrows_t.T.reshape(NS, CPT, E), dst)
        acc = (acc3.reshape(NC, NS, NPAD, CPT)
               .transpose(0, 2, 1, 3).reshape(NC, NPAD, ACCW)[:, :N])
        h = _finalize(acc[0], acc[1],
                      bp["bias"].reshape(1, H * C),
                      bp["ln_g"].reshape(1, HID), bp["ln_b"].reshape(1, HID))

    nh = params["node_head"]
    eh = params["edge_head"]
    em = params["edge_mlp"]
    rc = params["recon"]
    w6p = jnp.pad(eh["W"], ((0, 0), (0, 16 - EDGE_CLS)))
    b6p = jnp.pad(eh["b"], (0, 16 - EDGE_CLS))
    zpad = jnp.zeros((HID, 112), jnp.float32)
    w_all = jnp.concatenate(
        [nh["W"], rc["W"], em["W1"][HID:], em["W1"][:HID], w6p, zpad],
        axis=1)
    b_all = jnp.concatenate(
        [nh["b"], rc["b"], jnp.zeros((HID,), jnp.float32), em["b1"], b6p,
         jnp.zeros((112,), jnp.float32)]).reshape(1, -1)
    node_preds, recon, v, uz = _node_head(h, w_all, b_all)

    guz, gv = _sc_gather([uz, v], [src, dst])
    edge_preds, exist = _head_edge(guz, gv, em["W2"].reshape(1, HID),
                                   em["b2"].reshape(1, 1))

    return (node_preds, edge_preds, exist, recon)


# speedup vs baseline: 1.0730x; 1.0730x over previous
import functools

import jax
import jax.numpy as jnp
from jax import lax
from jax.experimental import pallas as pl
from jax.experimental.pallas import tpu as pltpu
from jax.experimental.pallas import tpu_sc as plsc

N = 10000
E = 320000
D = 128
HID = 128
H = 8
C = 16
NODE_CLS = 8
EDGE_CLS = 6

NC = 2
NS = 16
NW = NC * NS
EPW = E // NW
GCH = 400
GSUB = 80
NPAD = 10240
ACCW = 144
CPT = ACCW // NS
SCH = 640
EPC = E // NC

_mesh = plsc.VectorSubcoreMesh(
    core_axis_name="c", subcore_axis_name="s", num_cores=NC, num_subcores=NS)


def _full_spec(shape):
    nd = len(shape)
    return pl.BlockSpec(shape, lambda *_i, _nd=nd: (0,) * _nd)


def _mm_bias(a, w, b, blk):
    r, k0 = a.shape
    k = w.shape[1]

    def body(a_r, w_r, b_r, o_r):
        o_r[...] = (
            jnp.dot(a_r[...], w_r[...], preferred_element_type=jnp.float32)
            + b_r[...])

    return pl.pallas_call(
        body,
        grid=(r // blk,),
        in_specs=[
            pl.BlockSpec((blk, k0), lambda i: (i, 0)),
            _full_spec(w.shape),
            _full_spec(b.shape),
        ],
        out_specs=pl.BlockSpec((blk, k), lambda i: (i, 0)),
        out_shape=jax.ShapeDtypeStruct((r, k), jnp.float32),
    )(a, w, b)


def _fold_weights(wup, bup, we1, we2):
    def body(wu, bu, w1, w2, m1, m2, c1, c2):
        wuv = wu[...]
        buv = bu[...]
        m1[...] = jnp.dot(wuv, w1[...], preferred_element_type=jnp.float32)
        m2[...] = jnp.dot(wuv, w2[...], preferred_element_type=jnp.float32)
        c1[...] = jnp.dot(buv, w1[...], preferred_element_type=jnp.float32)
        c2[...] = jnp.dot(buv, w2[...], preferred_element_type=jnp.float32)

    return pl.pallas_call(
        body,
        in_specs=[_full_spec(wup.shape), _full_spec(bup.shape),
                  _full_spec(we1.shape), _full_spec(we2.shape)],
        out_specs=[_full_spec((HID, HID)), _full_spec((HID, HID)),
                   _full_spec((1, HID)), _full_spec((1, HID))],
        out_shape=[jax.ShapeDtypeStruct((HID, HID), jnp.float32),
                   jax.ShapeDtypeStruct((HID, HID), jnp.float32),
                   jax.ShapeDtypeStruct((1, HID), jnp.float32),
                   jax.ShapeDtypeStruct((1, HID), jnp.float32)],
    )(wup, bup, we1, we2)


def _gat_edge(ef, g1, g2, m_w, cb, att_a, blk=3200):
    def body(ef_r, g1_r, g2_r, mw_r, cb_r, a_r, o_r):
        g1v = g1_r[...]
        eap = (jnp.dot(ef_r[...], mw_r[...], preferred_element_type=jnp.float32)
               + cb_r[...])
        m = g1v + g2_r[...] + eap
        m = jnp.maximum(m, 0.2 * m)
        logits = jnp.dot(m, a_r[...], preferred_element_type=jnp.float32)
        e = jnp.exp(jnp.minimum(logits, 75.0))
        parts = [g1v[:, 16 * h:16 * (h + 1)] * e[:, h:h + 1] for h in range(H)]
        parts.append(e)
        parts.append(jnp.zeros((blk, ACCW - 136), jnp.float32))
        o_r[...] = jnp.concatenate(parts, axis=1).T

    return pl.pallas_call(
        body,
        grid=(E // blk,),
        in_specs=[
            pl.BlockSpec((blk, D), lambda i: (i, 0)),
            pl.BlockSpec((blk, HID), lambda i: (i, 0)),
            pl.BlockSpec((blk, HID), lambda i: (i, 0)),
            _full_spec(m_w.shape),
            _full_spec(cb.shape),
            _full_spec(att_a.shape),
        ],
        out_specs=pl.BlockSpec((ACCW, blk), lambda i: (0, i)),
        out_shape=jax.ShapeDtypeStruct((ACCW, E), jnp.float32),
    )(ef, g1, g2, m_w, cb, att_a)


def _finalize(a0, a1, bias, ln_g, ln_b, blk=2000):
    def body(a0_r, a1_r, b_r, g_r, bb_r, o_r):
        t = a0_r[...] + a1_r[...]
        cols = [t[:, 16 * h:16 * (h + 1)] / (t[:, 128 + h:129 + h] + 1e-16)
                for h in range(H)]
        o = jnp.concatenate(cols, axis=1) + b_r[...]
        mu = jnp.mean(o, axis=1, keepdims=True)
        var = jnp.mean((o - mu) * (o - mu), axis=1, keepdims=True)
        o = (o - mu) * lax.rsqrt(var + 1e-5) * g_r[...] + bb_r[...]
        o_r[...] = jnp.maximum(o, 0.0)

    return pl.pallas_call(
        body,
        grid=(N // blk,),
        in_specs=[
            pl.BlockSpec((blk, ACCW), lambda i: (i, 0)),
            pl.BlockSpec((blk, ACCW), lambda i: (i, 0)),
            _full_spec(bias.shape),
            _full_spec(ln_g.shape),
            _full_spec(ln_b.shape),
        ],
        out_specs=pl.BlockSpec((blk, HID), lambda i: (i, 0)),
        out_shape=jax.ShapeDtypeStruct((N, HID), jnp.float32),
    )(a0, a1, bias, ln_g, ln_b)


def _node_head(h, w_all, b_all, blk=2000):
    k = w_all.shape[1]

    def body(h_r, w_r, b_r, o_np, o_rec, o_v, o_uz):
        z = (jnp.dot(h_r[...], w_r[...], preferred_element_type=jnp.float32)
             + b_r[...])
        zn = z[:, :8]
        zn = zn - jnp.max(zn, axis=1, keepdims=True)
        ez = jnp.exp(zn)
        o_np[...] = ez / jnp.sum(ez, axis=1, keepdims=True)
        o_rec[...] = z[:, 8:136]
        o_v[...] = z[:, 136:264]
        o_uz[...] = z[:, 264:520]

    return pl.pallas_call(
        body,
        grid=(N // blk,),
        in_specs=[
            pl.BlockSpec((blk, HID), lambda i: (i, 0)),
            _full_spec(w_all.shape),
            _full_spec(b_all.shape),
        ],
        out_specs=[
            pl.BlockSpec((blk, NODE_CLS), lambda i: (i, 0)),
            pl.BlockSpec((blk, D), lambda i: (i, 0)),
            pl.BlockSpec((blk, HID), lambda i: (i, 0)),
            pl.BlockSpec((blk, 256), lambda i: (i, 0)),
        ],
        out_shape=[
            jax.ShapeDtypeStruct((N, NODE_CLS), jnp.float32),
            jax.ShapeDtypeStruct((N, D), jnp.float32),
            jax.ShapeDtypeStruct((N, HID), jnp.float32),
            jax.ShapeDtypeStruct((N, 256), jnp.float32),
        ],
    )(h, w_all, b_all)


def _head_edge(guz, gv, w2row, b2, blk=2000):
    def body(guz_r, gv_r, w2_r, b2_r, o_ep, o_ex):
        guz = guz_r[...]
        zz = guz[:, 128:128 + EDGE_CLS]
        zz = zz - jnp.max(zz, axis=1, keepdims=True)
        ez = jnp.exp(zz)
        o_ep[...] = ez / jnp.sum(ez, axis=1, keepdims=True)
        hid = jnp.maximum(guz[:, :128] + gv_r[...], 0.0)
        s = jnp.sum(hid * w2_r[...], axis=1, keepdims=True) + b2_r[...]
        o_ex[...] = 1.0 / (1.0 + jnp.exp(-s))

    return pl.pallas_call(
        body,
        grid=(E // blk,),
        in_specs=[
            pl.BlockSpec((blk, 256), lambda i: (i, 0)),
            pl.BlockSpec((blk, HID), lambda i: (i, 0)),
            _full_spec(w2row.shape),
            _full_spec(b2.shape),
        ],
        out_specs=[
            pl.BlockSpec((blk, EDGE_CLS), lambda i: (i, 0)),
            pl.BlockSpec((blk, 1), lambda i: (i, 0)),
        ],
        out_shape=[
            jax.ShapeDtypeStruct((E, EDGE_CLS), jnp.float32),
            jax.ShapeDtypeStruct((E, 1), jnp.float32),
        ],
    )(guz, gv, w2row, b2)


def _sc_gather(tables, idxs, gch=80):
    nt = len(tables)
    ks = [int(t.shape[1]) for t in tables]
    out_type = tuple(
        jax.ShapeDtypeStruct((E, k), jnp.float32) for k in ks)
    scratch = []
    for k in ks:
        scratch += [
            pltpu.VMEM((gch,), jnp.int32),
            pltpu.VMEM((gch,), jnp.int32),
            pltpu.VMEM((gch, k), jnp.float32),
            pltpu.VMEM((gch, k), jnp.float32),
            pltpu.SemaphoreType.DMA,
            pltpu.SemaphoreType.DMA,
        ]
    nch = EPW // gch

    def body(*refs):
        tabs = refs[:nt]
        idxr = refs[nt:2 * nt]
        outs = refs[2 * nt:3 * nt]
        scr = refs[3 * nt:]
        ivs = [(scr[6 * t], scr[6 * t + 1]) for t in range(nt)]
        rvs = [(scr[6 * t + 2], scr[6 * t + 3]) for t in range(nt)]
        sems = [(scr[6 * t + 4], scr[6 * t + 5]) for t in range(nt)]
        cid = lax.axis_index("c")
        sid = lax.axis_index("s")
        wid = sid * NC + cid

        def base(i):
            return pl.multiple_of(wid * EPW + i * gch, gch)

        def stage(i, b):
            for t in range(nt):
                pltpu.sync_copy(idxr[t].at[pl.ds(base(i), gch)], ivs[t][b])
                pltpu.async_copy(tabs[t].at[ivs[t][b]], rvs[t][b],
                                 sems[t][b])

        def drain(i, b):
            for t in range(nt):
                pltpu.make_async_copy(tabs[t].at[ivs[t][b]], rvs[t][b],
                                      sems[t][b]).wait()
                pltpu.sync_copy(rvs[t][b], outs[t].at[pl.ds(base(i), gch)])

        stage(0, 0)

        def pairstep(ii, carry):
            i1 = 2 * ii + 1
            stage(i1, 1)
            drain(i1 - 1, 0)
            stage(i1 + 1, 0)
            drain(i1, 1)
            return carry

        lax.fori_loop(0, (nch - 1) // 2, pairstep, 0)
        drain(nch - 1, 0)

    fn = pl.kernel(body, out_type=out_type, mesh=_mesh,
                   scratch_types=tuple(scratch))
    return fn(*tables, *idxs)


def _sc_scatter_add(rows_t, dst):
    out_type = jax.ShapeDtypeStruct((NC, NS, NPAD * CPT), jnp.float32)
    scratch = [
        pltpu.VMEM((CPT, SCH), jnp.float32),
        pltpu.VMEM((CPT, SCH), jnp.float32),
        pltpu.VMEM((SCH,), jnp.int32),
        pltpu.VMEM((SCH,), jnp.int32),
        pltpu.VMEM((NPAD * CPT,), jnp.float32),
        pltpu.SemaphoreType.DMA,
        pltpu.SemaphoreType.DMA,
    ]
    nch = EPC // SCH

    def body(rows_r, dst_r, out_r, rv0, rv1, dv0, dv1, acc, sem0, sem1):
        cid = lax.axis_index("c")
        sid = lax.axis_index("s")
        rvs, dvs, sems = (rv0, rv1), (dv0, dv1), (sem0, sem1)

        def start(i, b):
            eb = pl.multiple_of(cid * EPC + i * SCH, SCH)
            pltpu.async_copy(rows_r.at[sid, :, pl.ds(eb, SCH)], rvs[b],
                             sems[b])
            pltpu.async_copy(dst_r.at[pl.ds(eb, SCH)], dvs[b], sems[b])

        def wait(b):
            pltpu.make_async_copy(rows_r.at[sid, :, pl.ds(0, SCH)], rvs[b],
                                  sems[b]).wait()
            pltpu.make_async_copy(dst_r.at[pl.ds(0, SCH)], dvs[b],
                                  sems[b]).wait()

        def compute(b):
            rv, dv = rvs[b], dvs[b]

            def group(g, carry2):
                dstv = dv[pl.ds(g * 16, 16)]
                lane = lax.iota(jnp.int32, 16)
                base_idx = dstv * CPT
                for k in range(CPT):
                    plsc.addupdate_scatter(acc, [base_idx + k],
                                           rv[k, pl.ds(g * 16, 16)],
                                           mask=lane >= 0)
                return carry2

            lax.fori_loop(0, SCH // 16, group, 0)

        def zstep(i, carry):
            acc[pl.ds(i * 16, 16)] = jnp.zeros((16,), jnp.float32)
            return carry

        start(0, 0)
        lax.fori_loop(0, NPAD * CPT // 16, zstep, 0)

        def step(ii, carry):
            i0 = 2 * ii
            start(i0 + 1, 1)
            wait(0)
            compute(0)
            start(lax.rem(i0 + 2, nch), 0)
            wait(1)
            compute(1)
            return carry

        lax.fori_loop(0, nch // 2, step, 0)
        wait(0)
        pltpu.sync_copy(acc, out_r.at[cid, sid])

    fn = pl.kernel(
        body, out_type=out_type, mesh=_mesh, scratch_types=scratch,
        compiler_params=pltpu.CompilerParams(needs_layout_passes=False))
    return fn(rows_t, dst)


def _att_matrix(att):
    eye = jnp.eye(H, dtype=jnp.float32)
    return (att[:, :, None] * eye[:, None, :]).reshape(H * C, H)


def kernel(x, edge_features, edge_index, params):
    src = edge_index[0]
    dst = edge_index[1]

    up = params["up"]
    wup = up["W"]
    bup = up["b"].reshape(1, HID)
    blk0, blk1 = params["blocks"]

    m1, m2, cb1, cb2 = _fold_weights(wup, bup, blk0["We"], blk1["We"])

    h = _mm_bias(x, wup, bup, 2000)

    for bp, m_w, cb in ((blk0, m1, cb1), (blk1, m2, cb2)):
        xl = _mm_bias(h, bp["Wl"], bp["bl"].reshape(1, H * C), 2000)
        xr = _mm_bias(h, bp["Wr"], bp["br"].reshape(1, H * C), 2000)
        g1, g2 = _sc_gather([xl, xr], [src, dst])
        wrows_t = _gat_edge(edge_features, g1, g2, m_w, cb,
                            _att_matrix(bp["att"]))
        acc3 = _sc_scatter_add(wrows_t.reshape(NS, CPT, E), dst)
        acc = (acc3.reshape(NC, NS, NPAD, CPT)
               .transpose(0, 2, 1, 3).reshape(NC, NPAD, ACCW)[:, :N])
        h = _finalize(acc[0], acc[1],
                      bp["bias"].reshape(1, H * C),
                      bp["ln_g"].reshape(1, HID), bp["ln_b"].reshape(1, HID))

    nh = params["node_head"]
    eh = params["edge_head"]
    em = params["edge_mlp"]
    rc = params["recon"]
    w6p = jnp.pad(eh["W"], ((0, 0), (0, 16 - EDGE_CLS)))
    b6p = jnp.pad(eh["b"], (0, 16 - EDGE_CLS))
    zpad = jnp.zeros((HID, 112), jnp.float32)
    w_all = jnp.concatenate(
        [nh["W"], rc["W"], em["W1"][HID:], em["W1"][:HID], w6p, zpad],
        axis=1)
    b_all = jnp.concatenate(
        [nh["b"], rc["b"], jnp.zeros((HID,), jnp.float32), em["b1"], b6p,
         jnp.zeros((112,), jnp.float32)]).reshape(1, -1)
    node_preds, recon, v, uz = _node_head(h, w_all, b_all)

    guz, gv = _sc_gather([uz, v], [src, dst])
    edge_preds, exist = _head_edge(guz, gv, em["W2"].reshape(1, HID),
                                   em["b2"].reshape(1, 1))

    return (node_preds, edge_preds, exist, recon)


# speedup vs baseline: 1.0733x; 1.0003x over previous
import functools

import jax
import jax.numpy as jnp
from jax import lax
from jax.experimental import pallas as pl
from jax.experimental.pallas import tpu as pltpu
from jax.experimental.pallas import tpu_sc as plsc

N = 10000
E = 320000
D = 128
HID = 128
H = 8
C = 16
NODE_CLS = 8
EDGE_CLS = 6

NC = 2
NS = 16
NW = NC * NS
EPW = E // NW
GCH = 400
GSUB = 80
NPAD = 10240
ACCW = 144
CPT = ACCW // NS
SCH = 640
EPC = E // NC

_mesh = plsc.VectorSubcoreMesh(
    core_axis_name="c", subcore_axis_name="s", num_cores=NC, num_subcores=NS)


def _full_spec(shape):
    nd = len(shape)
    return pl.BlockSpec(shape, lambda *_i, _nd=nd: (0,) * _nd)


def _mm_bias(a, w, b, blk):
    r, k0 = a.shape
    k = w.shape[1]

    def body(a_r, w_r, b_r, o_r):
        o_r[...] = (
            jnp.dot(a_r[...], w_r[...], preferred_element_type=jnp.float32)
            + b_r[...])

    return pl.pallas_call(
        body,
        grid=(r // blk,),
        in_specs=[
            pl.BlockSpec((blk, k0), lambda i: (i, 0)),
            _full_spec(w.shape),
            _full_spec(b.shape),
        ],
        out_specs=pl.BlockSpec((blk, k), lambda i: (i, 0)),
        out_shape=jax.ShapeDtypeStruct((r, k), jnp.float32),
    )(a, w, b)


def _fold_weights(wup, bup, we1, we2):
    def body(wu, bu, w1, w2, m1, m2, c1, c2):
        wuv = wu[...]
        buv = bu[...]
        m1[...] = jnp.dot(wuv, w1[...], preferred_element_type=jnp.float32)
        m2[...] = jnp.dot(wuv, w2[...], preferred_element_type=jnp.float32)
        c1[...] = jnp.dot(buv, w1[...], preferred_element_type=jnp.float32)
        c2[...] = jnp.dot(buv, w2[...], preferred_element_type=jnp.float32)

    return pl.pallas_call(
        body,
        in_specs=[_full_spec(wup.shape), _full_spec(bup.shape),
                  _full_spec(we1.shape), _full_spec(we2.shape)],
        out_specs=[_full_spec((HID, HID)), _full_spec((HID, HID)),
                   _full_spec((1, HID)), _full_spec((1, HID))],
        out_shape=[jax.ShapeDtypeStruct((HID, HID), jnp.float32),
                   jax.ShapeDtypeStruct((HID, HID), jnp.float32),
                   jax.ShapeDtypeStruct((1, HID), jnp.float32),
                   jax.ShapeDtypeStruct((1, HID), jnp.float32)],
    )(wup, bup, we1, we2)


def _gat_edge(ef, g1, g2, m_w, cb, att_a, blk=3200):
    def body(ef_r, g1_r, g2_r, mw_r, cb_r, a_r, o_r):
        g1v = g1_r[...]
        eap = (jnp.dot(ef_r[...], mw_r[...], preferred_element_type=jnp.float32)
               + cb_r[...])
        m = g1v + g2_r[...] + eap
        m = jnp.maximum(m, 0.2 * m)
        logits = jnp.dot(m, a_r[...], preferred_element_type=jnp.float32)
        e = jnp.exp(jnp.minimum(logits, 75.0))
        parts = [g1v[:, 16 * h:16 * (h + 1)] * e[:, h:h + 1] for h in range(H)]
        parts.append(e)
        parts.append(jnp.zeros((blk, ACCW - 136), jnp.float32))
        o_r[...] = jnp.concatenate(parts, axis=1).T

    return pl.pallas_call(
        body,
        grid=(E // blk,),
        in_specs=[
            pl.BlockSpec((blk, D), lambda i: (i, 0)),
            pl.BlockSpec((blk, HID), lambda i: (i, 0)),
            pl.BlockSpec((blk, HID), lambda i: (i, 0)),
            _full_spec(m_w.shape),
            _full_spec(cb.shape),
            _full_spec(att_a.shape),
        ],
        out_specs=pl.BlockSpec((ACCW, blk), lambda i: (0, i)),
        out_shape=jax.ShapeDtypeStruct((ACCW, E), jnp.float32),
    )(ef, g1, g2, m_w, cb, att_a)


def _finalize(a0, a1, bias, ln_g, ln_b, blk=2000):
    def body(a0_r, a1_r, b_r, g_r, bb_r, o_r):
        t = a0_r[...] + a1_r[...]
        cols = [t[:, 16 * h:16 * (h + 1)] / (t[:, 128 + h:129 + h] + 1e-16)
                for h in range(H)]
        o = jnp.concatenate(cols, axis=1) + b_r[...]
        mu = jnp.mean(o, axis=1, keepdims=True)
        var = jnp.mean((o - mu) * (o - mu), axis=1, keepdims=True)
        o = (o - mu) * lax.rsqrt(var + 1e-5) * g_r[...] + bb_r[...]
        o_r[...] = jnp.maximum(o, 0.0)

    return pl.pallas_call(
        body,
        grid=(N // blk,),
        in_specs=[
            pl.BlockSpec((blk, ACCW), lambda i: (i, 0)),
            pl.BlockSpec((blk, ACCW), lambda i: (i, 0)),
            _full_spec(bias.shape),
            _full_spec(ln_g.shape),
            _full_spec(ln_b.shape),
        ],
        out_specs=pl.BlockSpec((blk, HID), lambda i: (i, 0)),
        out_shape=jax.ShapeDtypeStruct((N, HID), jnp.float32),
    )(a0, a1, bias, ln_g, ln_b)


def _node_head(h, w_all, b_all, blk=2000):
    k = w_all.shape[1]

    def body(h_r, w_r, b_r, o_np, o_rec, o_v, o_uz):
        z = (jnp.dot(h_r[...], w_r[...], preferred_element_type=jnp.float32)
             + b_r[...])
        zn = z[:, :8]
        zn = zn - jnp.max(zn, axis=1, keepdims=True)
        ez = jnp.exp(zn)
        o_np[...] = ez / jnp.sum(ez, axis=1, keepdims=True)
        o_rec[...] = z[:, 8:136]
        o_v[...] = z[:, 136:264]
        o_uz[...] = z[:, 264:520]

    return pl.pallas_call(
        body,
        grid=(N // blk,),
        in_specs=[
            pl.BlockSpec((blk, HID), lambda i: (i, 0)),
            _full_spec(w_all.shape),
            _full_spec(b_all.shape),
        ],
        out_specs=[
            pl.BlockSpec((blk, NODE_CLS), lambda i: (i, 0)),
            pl.BlockSpec((blk, D), lambda i: (i, 0)),
            pl.BlockSpec((blk, HID), lambda i: (i, 0)),
            pl.BlockSpec((blk, 256), lambda i: (i, 0)),
        ],
        out_shape=[
            jax.ShapeDtypeStruct((N, NODE_CLS), jnp.float32),
            jax.ShapeDtypeStruct((N, D), jnp.float32),
            jax.ShapeDtypeStruct((N, HID), jnp.float32),
            jax.ShapeDtypeStruct((N, 256), jnp.float32),
        ],
    )(h, w_all, b_all)


def _head_edge(guz, gv, w2row, b2, blk=2000):
    def body(guz_r, gv_r, w2_r, b2_r, o_ep, o_ex):
        guz = guz_r[...]
        zz = guz[:, 128:128 + EDGE_CLS]
        zz = zz - jnp.max(zz, axis=1, keepdims=True)
        ez = jnp.exp(zz)
        o_ep[...] = ez / jnp.sum(ez, axis=1, keepdims=True)
        hid = jnp.maximum(guz[:, :128] + gv_r[...], 0.0)
        s = jnp.sum(hid * w2_r[...], axis=1, keepdims=True) + b2_r[...]
        o_ex[...] = 1.0 / (1.0 + jnp.exp(-s))

    return pl.pallas_call(
        body,
        grid=(E // blk,),
        in_specs=[
            pl.BlockSpec((blk, 256), lambda i: (i, 0)),
            pl.BlockSpec((blk, HID), lambda i: (i, 0)),
            _full_spec(w2row.shape),
            _full_spec(b2.shape),
        ],
        out_specs=[
            pl.BlockSpec((blk, EDGE_CLS), lambda i: (i, 0)),
            pl.BlockSpec((blk, 1), lambda i: (i, 0)),
        ],
        out_shape=[
            jax.ShapeDtypeStruct((E, EDGE_CLS), jnp.float32),
            jax.ShapeDtypeStruct((E, 1), jnp.float32),
        ],
    )(guz, gv, w2row, b2)


def _sc_gather(tables, idxs, gch=80):
    nt = len(tables)
    ks = [int(t.shape[1]) for t in tables]
    out_type = tuple(
        jax.ShapeDtypeStruct((E, k), jnp.float32) for k in ks)
    scratch = []
    for k in ks:
        scratch += [
            pltpu.VMEM((gch,), jnp.int32),
            pltpu.VMEM((gch,), jnp.int32),
            pltpu.VMEM((gch, k), jnp.float32),
            pltpu.VMEM((gch, k), jnp.float32),
            pltpu.SemaphoreType.DMA,
            pltpu.SemaphoreType.DMA,
        ]
    nch = EPW // gch

    def body(*refs):
        tabs = refs[:nt]
        idxr = refs[nt:2 * nt]
        outs = refs[2 * nt:3 * nt]
        scr = refs[3 * nt:]
        ivs = [(scr[6 * t], scr[6 * t + 1]) for t in range(nt)]
        rvs = [(scr[6 * t + 2], scr[6 * t + 3]) for t in range(nt)]
        sems = [(scr[6 * t + 4], scr[6 * t + 5]) for t in range(nt)]
        cid = lax.axis_index("c")
        sid = lax.axis_index("s")
        wid = sid * NC + cid

        def base(i):
            return pl.multiple_of(wid * EPW + i * gch, gch)

        def stage(i, b):
            for t in range(nt):
                pltpu.sync_copy(idxr[t].at[pl.ds(base(i), gch)], ivs[t][b])
                pltpu.async_copy(tabs[t].at[ivs[t][b]], rvs[t][b],
                                 sems[t][b])

        def drain(i, b):
            for t in range(nt):
                pltpu.make_async_copy(tabs[t].at[ivs[t][b]], rvs[t][b],
                                      sems[t][b]).wait()
                pltpu.sync_copy(rvs[t][b], outs[t].at[pl.ds(base(i), gch)])

        stage(0, 0)

        def pairstep(ii, carry):
            i1 = 2 * ii + 1
            stage(i1, 1)
            drain(i1 - 1, 0)
            stage(i1 + 1, 0)
            drain(i1, 1)
            return carry

        lax.fori_loop(0, (nch - 1) // 2, pairstep, 0)
        drain(nch - 1, 0)

    fn = pl.kernel(body, out_type=out_type, mesh=_mesh,
                   scratch_types=tuple(scratch))
    return fn(*tables, *idxs)


def _sc_scatter_add(rows_t, dst):
    out_type = jax.ShapeDtypeStruct((NC, NS, NPAD * CPT), jnp.float32)
    scratch = [
        pltpu.VMEM((CPT, SCH), jnp.float32),
        pltpu.VMEM((CPT, SCH), jnp.float32),
        pltpu.VMEM((SCH,), jnp.int32),
        pltpu.VMEM((SCH,), jnp.int32),
        pltpu.VMEM((NPAD * CPT,), jnp.float32),
        pltpu.SemaphoreType.DMA,
        pltpu.SemaphoreType.DMA,
    ]
    nch = EPC // SCH

    def body(rows_r, dst_r, out_r, rv0, rv1, dv0, dv1, acc, sem0, sem1):
        cid = lax.axis_index("c")
        sid = lax.axis_index("s")
        rvs, dvs, sems = (rv0, rv1), (dv0, dv1), (sem0, sem1)

        def start(i, b):
            eb = pl.multiple_of(cid * EPC + i * SCH, SCH)
            pltpu.async_copy(rows_r.at[sid, :, pl.ds(eb, SCH)], rvs[b],
                             sems[b])
            pltpu.async_copy(dst_r.at[pl.ds(eb, SCH)], dvs[b], sems[b])

        def wait(b):
            pltpu.make_async_copy(rows_r.at[sid, :, pl.ds(0, SCH)], rvs[b],
                                  sems[b]).wait()
            pltpu.make_async_copy(dst_r.at[pl.ds(0, SCH)], dvs[b],
                                  sems[b]).wait()

        def compute(b):
            rv, dv = rvs[b], dvs[b]

            def group(g2, carry2):
                lane = lax.iota(jnp.int32, 16)
                for u in range(2):
                    off = (2 * g2 + u) * 16
                    dstv = dv[pl.ds(off, 16)]
                    base_idx = dstv * CPT
                    for k in range(CPT):
                        plsc.addupdate_scatter(acc, [base_idx + k],
                                               rv[k, pl.ds(off, 16)],
                                               mask=lane >= 0)
                return carry2

            lax.fori_loop(0, SCH // 32, group, 0)

        def zstep(i, carry):
            acc[pl.ds(i * 16, 16)] = jnp.zeros((16,), jnp.float32)
            return carry

        start(0, 0)
        lax.fori_loop(0, NPAD * CPT // 16, zstep, 0)

        def step(ii, carry):
            i0 = 2 * ii
            start(i0 + 1, 1)
            wait(0)
            compute(0)
            start(lax.rem(i0 + 2, nch), 0)
            wait(1)
            compute(1)
            return carry

        lax.fori_loop(0, nch // 2, step, 0)
        wait(0)
        pltpu.sync_copy(acc, out_r.at[cid, sid])

    fn = pl.kernel(
        body, out_type=out_type, mesh=_mesh, scratch_types=scratch,
        compiler_params=pltpu.CompilerParams(needs_layout_passes=False))
    return fn(rows_t, dst)


def _att_matrix(att):
    eye = jnp.eye(H, dtype=jnp.float32)
    return (att[:, :, None] * eye[:, None, :]).reshape(H * C, H)


def kernel(x, edge_features, edge_index, params):
    src = edge_index[0]
    dst = edge_index[1]

    up = params["up"]
    wup = up["W"]
    bup = up["b"].reshape(1, HID)
    blk0, blk1 = params["blocks"]

    m1, m2, cb1, cb2 = _fold_weights(wup, bup, blk0["We"], blk1["We"])

    h = _mm_bias(x, wup, bup, 2000)

    for bp, m_w, cb in ((blk0, m1, cb1), (blk1, m2, cb2)):
        xl = _mm_bias(h, bp["Wl"], bp["bl"].reshape(1, H * C), 2000)
        xr = _mm_bias(h, bp["Wr"], bp["br"].reshape(1, H * C), 2000)
        g1, g2 = _sc_gather([xl, xr], [src, dst])
        wrows_t = _gat_edge(edge_features, g1, g2, m_w, cb,
                            _att_matrix(bp["att"]))
        acc3 = _sc_scatter_add(wrows_t.reshape(NS, CPT, E), dst)
        acc = (acc3.reshape(NC, NS, NPAD, CPT)
               .transpose(0, 2, 1, 3).reshape(NC, NPAD, ACCW)[:, :N])
        h = _finalize(acc[0], acc[1],
                      bp["bias"].reshape(1, H * C),
                      bp["ln_g"].reshape(1, HID), bp["ln_b"].reshape(1, HID))

    nh = params["node_head"]
    eh = params["edge_head"]
    em = params["edge_mlp"]
    rc = params["recon"]
    w6p = jnp.pad(eh["W"], ((0, 0), (0, 16 - EDGE_CLS)))
    b6p = jnp.pad(eh["b"], (0, 16 - EDGE_CLS))
    zpad = jnp.zeros((HID, 112), jnp.float32)
    w_all = jnp.concatenate(
        [nh["W"], rc["W"], em["W1"][HID:], em["W1"][:HID], w6p, zpad],
        axis=1)
    b_all = jnp.concatenate(
        [nh["b"], rc["b"], jnp.zeros((HID,), jnp.float32), em["b1"], b6p,
         jnp.zeros((112,), jnp.float32)]).reshape(1, -1)
    node_preds, recon, v, uz = _node_head(h, w_all, b_all)

    guz, gv = _sc_gather([uz, v], [src, dst])
    edge_preds, exist = _head_edge(guz, gv, em["W2"].reshape(1, HID),
                                   em["b2"].reshape(1, 1))

    return (node_preds, edge_preds, exist, recon)


# speedup vs baseline: 1.0741x; 1.0007x over previous
import functools

import jax
import jax.numpy as jnp
from jax import lax
from jax.experimental import pallas as pl
from jax.experimental.pallas import tpu as pltpu
from jax.experimental.pallas import tpu_sc as plsc

N = 10000
E = 320000
D = 128
HID = 128
H = 8
C = 16
NODE_CLS = 8
EDGE_CLS = 6

NC = 2
NS = 16
NW = NC * NS
EPW = E // NW
GCH = 400
GSUB = 80
NPAD = 10240
ACCW = 144
CPT = ACCW // NS
SCH = 640
EPC = E // NC

@functools.cache
def _mesh():
    return plsc.VectorSubcoreMesh(
        core_axis_name="c", subcore_axis_name="s",
        num_cores=NC, num_subcores=NS)


def _full_spec(shape):
    nd = len(shape)
    return pl.BlockSpec(shape, lambda *_i, _nd=nd: (0,) * _nd)


def _mm_bias(a, w, b, blk):
    r, k0 = a.shape
    k = w.shape[1]

    def body(a_r, w_r, b_r, o_r):
        o_r[...] = (
            jnp.dot(a_r[...], w_r[...], preferred_element_type=jnp.float32)
            + b_r[...])

    return pl.pallas_call(
        body,
        grid=(r // blk,),
        in_specs=[
            pl.BlockSpec((blk, k0), lambda i: (i, 0)),
            _full_spec(w.shape),
            _full_spec(b.shape),
        ],
        out_specs=pl.BlockSpec((blk, k), lambda i: (i, 0)),
        out_shape=jax.ShapeDtypeStruct((r, k), jnp.float32),
    )(a, w, b)


def _fold_weights(wup, bup, we1, we2):
    def body(wu, bu, w1, w2, m1, m2, c1, c2):
        wuv = wu[...]
        buv = bu[...]
        m1[...] = jnp.dot(wuv, w1[...], preferred_element_type=jnp.float32)
        m2[...] = jnp.dot(wuv, w2[...], preferred_element_type=jnp.float32)
        c1[...] = jnp.dot(buv, w1[...], preferred_element_type=jnp.float32)
        c2[...] = jnp.dot(buv, w2[...], preferred_element_type=jnp.float32)

    return pl.pallas_call(
        body,
        in_specs=[_full_spec(wup.shape), _full_spec(bup.shape),
                  _full_spec(we1.shape), _full_spec(we2.shape)],
        out_specs=[_full_spec((HID, HID)), _full_spec((HID, HID)),
                   _full_spec((1, HID)), _full_spec((1, HID))],
        out_shape=[jax.ShapeDtypeStruct((HID, HID), jnp.float32),
                   jax.ShapeDtypeStruct((HID, HID), jnp.float32),
                   jax.ShapeDtypeStruct((1, HID), jnp.float32),
                   jax.ShapeDtypeStruct((1, HID), jnp.float32)],
    )(wup, bup, we1, we2)


def _gat_edge(ef, g1, g2, m_w, cb, att_a, blk=3200):
    def body(ef_r, g1_r, g2_r, mw_r, cb_r, a_r, o_r):
        g1v = g1_r[...]
        eap = (jnp.dot(ef_r[...], mw_r[...], preferred_element_type=jnp.float32)
               + cb_r[...])
        m = g1v + g2_r[...] + eap
        m = jnp.maximum(m, 0.2 * m)
        logits = jnp.dot(m, a_r[...], preferred_element_type=jnp.float32)
        e = jnp.exp(jnp.minimum(logits, 75.0))
        parts = [g1v[:, 16 * h:16 * (h + 1)] * e[:, h:h + 1] for h in range(H)]
        parts.append(e)
        parts.append(jnp.zeros((blk, ACCW - 136), jnp.float32))
        o_r[...] = jnp.concatenate(parts, axis=1).T

    return pl.pallas_call(
        body,
        grid=(E // blk,),
        in_specs=[
            pl.BlockSpec((blk, D), lambda i: (i, 0)),
            pl.BlockSpec((blk, HID), lambda i: (i, 0)),
            pl.BlockSpec((blk, HID), lambda i: (i, 0)),
            _full_spec(m_w.shape),
            _full_spec(cb.shape),
            _full_spec(att_a.shape),
        ],
        out_specs=pl.BlockSpec((ACCW, blk), lambda i: (0, i)),
        out_shape=jax.ShapeDtypeStruct((ACCW, E), jnp.float32),
    )(ef, g1, g2, m_w, cb, att_a)


def _finalize(a0, a1, bias, ln_g, ln_b, blk=2000):
    def body(a0_r, a1_r, b_r, g_r, bb_r, o_r):
        t = a0_r[...] + a1_r[...]
        cols = [t[:, 16 * h:16 * (h + 1)] / (t[:, 128 + h:129 + h] + 1e-16)
                for h in range(H)]
        o = jnp.concatenate(cols, axis=1) + b_r[...]
        mu = jnp.mean(o, axis=1, keepdims=True)
        var = jnp.mean((o - mu) * (o - mu), axis=1, keepdims=True)
        o = (o - mu) * lax.rsqrt(var + 1e-5) * g_r[...] + bb_r[...]
        o_r[...] = jnp.maximum(o, 0.0)

    return pl.pallas_call(
        body,
        grid=(N // blk,),
        in_specs=[
            pl.BlockSpec((blk, ACCW), lambda i: (i, 0)),
            pl.BlockSpec((blk, ACCW), lambda i: (i, 0)),
            _full_spec(bias.shape),
            _full_spec(ln_g.shape),
            _full_spec(ln_b.shape),
        ],
        out_specs=pl.BlockSpec((blk, HID), lambda i: (i, 0)),
        out_shape=jax.ShapeDtypeStruct((N, HID), jnp.float32),
    )(a0, a1, bias, ln_g, ln_b)


def _node_head(h, w_all, b_all, blk=2000):
    k = w_all.shape[1]

    def body(h_r, w_r, b_r, o_np, o_rec, o_v, o_uz):
        z = (jnp.dot(h_r[...], w_r[...], preferred_element_type=jnp.float32)
             + b_r[...])
        zn = z[:, :8]
        zn = zn - jnp.max(zn, axis=1, keepdims=True)
        ez = jnp.exp(zn)
        o_np[...] = ez / jnp.sum(ez, axis=1, keepdims=True)
        o_rec[...] = z[:, 8:136]
        o_v[...] = z[:, 136:264]
        o_uz[...] = z[:, 264:520]

    return pl.pallas_call(
        body,
        grid=(N // blk,),
        in_specs=[
            pl.BlockSpec((blk, HID), lambda i: (i, 0)),
            _full_spec(w_all.shape),
            _full_spec(b_all.shape),
        ],
        out_specs=[
            pl.BlockSpec((blk, NODE_CLS), lambda i: (i, 0)),
            pl.BlockSpec((blk, D), lambda i: (i, 0)),
            pl.BlockSpec((blk, HID), lambda i: (i, 0)),
            pl.BlockSpec((blk, 256), lambda i: (i, 0)),
        ],
        out_shape=[
            jax.ShapeDtypeStruct((N, NODE_CLS), jnp.float32),
            jax.ShapeDtypeStruct((N, D), jnp.float32),
            jax.ShapeDtypeStruct((N, HID), jnp.float32),
            jax.ShapeDtypeStruct((N, 256), jnp.float32),
        ],
    )(h, w_all, b_all)


def _head_edge(guz, gv, w2row, b2, blk=2000):
    def body(guz_r, gv_r, w2_r, b2_r, o_ep, o_ex):
        guz = guz_r[...]
        zz = guz[:, 128:128 + EDGE_CLS]
        zz = zz - jnp.max(zz, axis=1, keepdims=True)
        ez = jnp.exp(zz)
        o_ep[...] = ez / jnp.sum(ez, axis=1, keepdims=True)
        hid = jnp.maximum(guz[:, :128] + gv_r[...], 0.0)
        s = jnp.sum(hid * w2_r[...], axis=1, keepdims=True) + b2_r[...]
        o_ex[...] = 1.0 / (1.0 + jnp.exp(-s))

    return pl.pallas_call(
        body,
        grid=(E // blk,),
        in_specs=[
            pl.BlockSpec((blk, 256), lambda i: (i, 0)),
            pl.BlockSpec((blk, HID), lambda i: (i, 0)),
            _full_spec(w2row.shape),
            _full_spec(b2.shape),
        ],
        out_specs=[
            pl.BlockSpec((blk, EDGE_CLS), lambda i: (i, 0)),
            pl.BlockSpec((blk, 1), lambda i: (i, 0)),
        ],
        out_shape=[
            jax.ShapeDtypeStruct((E, EDGE_CLS), jnp.float32),
            jax.ShapeDtypeStruct((E, 1), jnp.float32),
        ],
    )(guz, gv, w2row, b2)


def _sc_gather(tables, idxs, gch=80):
    nt = len(tables)
    ks = [int(t.shape[1]) for t in tables]
    out_type = tuple(
        jax.ShapeDtypeStruct((E, k), jnp.float32) for k in ks)
    scratch = []
    for k in ks:
        scratch += [
            pltpu.VMEM((gch,), jnp.int32),
            pltpu.VMEM((gch,), jnp.int32),
            pltpu.VMEM((gch, k), jnp.float32),
            pltpu.VMEM((gch, k), jnp.float32),
            pltpu.SemaphoreType.DMA,
            pltpu.SemaphoreType.DMA,
        ]
    nch = EPW // gch

    def body(*refs):
        tabs = refs[:nt]
        idxr = refs[nt:2 * nt]
        outs = refs[2 * nt:3 * nt]
        scr = refs[3 * nt:]
        ivs = [(scr[6 * t], scr[6 * t + 1]) for t in range(nt)]
        rvs = [(scr[6 * t + 2], scr[6 * t + 3]) for t in range(nt)]
        sems = [(scr[6 * t + 4], scr[6 * t + 5]) for t in range(nt)]
        cid = lax.axis_index("c")
        sid = lax.axis_index("s")
        wid = sid * NC + cid

        def base(i):
            return pl.multiple_of(wid * EPW + i * gch, gch)

        def stage(i, b):
            for t in range(nt):
                pltpu.sync_copy(idxr[t].at[pl.ds(base(i), gch)], ivs[t][b])
                pltpu.async_copy(tabs[t].at[ivs[t][b]], rvs[t][b],
                                 sems[t][b])

        def drain(i, b):
            for t in range(nt):
                pltpu.make_async_copy(tabs[t].at[ivs[t][b]], rvs[t][b],
                                      sems[t][b]).wait()
                pltpu.sync_copy(rvs[t][b], outs[t].at[pl.ds(base(i), gch)])

        stage(0, 0)

        def pairstep(ii, carry):
            i1 = 2 * ii + 1
            stage(i1, 1)
            drain(i1 - 1, 0)
            stage(i1 + 1, 0)
            drain(i1, 1)
            return carry

        lax.fori_loop(0, (nch - 1) // 2, pairstep, 0)
        drain(nch - 1, 0)

    fn = pl.kernel(body, out_type=out_type, mesh=_mesh(),
                   scratch_types=tuple(scratch))
    return fn(*tables, *idxs)


def _sc_scatter_add(rows_t, dst):
    out_type = jax.ShapeDtypeStruct((NC, NS, NPAD * CPT), jnp.float32)
    scratch = [
        pltpu.VMEM((CPT, SCH), jnp.float32),
        pltpu.VMEM((CPT, SCH), jnp.float32),
        pltpu.VMEM((SCH,), jnp.int32),
        pltpu.VMEM((SCH,), jnp.int32),
        pltpu.VMEM((NPAD * CPT,), jnp.float32),
        pltpu.SemaphoreType.DMA,
        pltpu.SemaphoreType.DMA,
    ]
    nch = EPC // SCH

    def body(rows_r, dst_r, out_r, rv0, rv1, dv0, dv1, acc, sem0, sem1):
        cid = lax.axis_index("c")
        sid = lax.axis_index("s")
        rvs, dvs, sems = (rv0, rv1), (dv0, dv1), (sem0, sem1)

        def start(i, b):
            eb = pl.multiple_of(cid * EPC + i * SCH, SCH)
            pltpu.async_copy(rows_r.at[sid, :, pl.ds(eb, SCH)], rvs[b],
                             sems[b])
            pltpu.async_copy(dst_r.at[pl.ds(eb, SCH)], dvs[b], sems[b])

        def wait(b):
            pltpu.make_async_copy(rows_r.at[sid, :, pl.ds(0, SCH)], rvs[b],
                                  sems[b]).wait()
            pltpu.make_async_copy(dst_r.at[pl.ds(0, SCH)], dvs[b],
                                  sems[b]).wait()

        def compute(b):
            rv, dv = rvs[b], dvs[b]

            def group(g2, carry2):
                lane = lax.iota(jnp.int32, 16)
                for u in range(2):
                    off = (2 * g2 + u) * 16
                    dstv = dv[pl.ds(off, 16)]
                    base_idx = dstv * CPT
                    for k in range(CPT):
                        plsc.addupdate_scatter(acc, [base_idx + k],
                                               rv[k, pl.ds(off, 16)],
                                               mask=lane >= 0)
                return carry2

            lax.fori_loop(0, SCH // 32, group, 0)

        def zstep(i, carry):
            acc[pl.ds(i * 16, 16)] = jnp.zeros((16,), jnp.float32)
            return carry

        start(0, 0)
        lax.fori_loop(0, NPAD * CPT // 16, zstep, 0)

        def step(ii, carry):
            i0 = 2 * ii
            start(i0 + 1, 1)
            wait(0)
            compute(0)
            start(lax.rem(i0 + 2, nch), 0)
            wait(1)
            compute(1)
            return carry

        lax.fori_loop(0, nch // 2, step, 0)
        wait(0)
        pltpu.sync_copy(acc, out_r.at[cid, sid])

    fn = pl.kernel(
        body, out_type=out_type, mesh=_mesh(), scratch_types=scratch,
        compiler_params=pltpu.CompilerParams(needs_layout_passes=False))
    return fn(rows_t, dst)


def _att_matrix(att):
    eye = jnp.eye(H, dtype=jnp.float32)
    return (att[:, :, None] * eye[:, None, :]).reshape(H * C, H)


def kernel(x, edge_features, edge_index, params):
    src = edge_index[0]
    dst = edge_index[1]

    up = params["up"]
    wup = up["W"]
    bup = up["b"].reshape(1, HID)
    blk0, blk1 = params["blocks"]

    m1, m2, cb1, cb2 = _fold_weights(wup, bup, blk0["We"], blk1["We"])

    h = _mm_bias(x, wup, bup, 2000)

    for bp, m_w, cb in ((blk0, m1, cb1), (blk1, m2, cb2)):
        xl = _mm_bias(h, bp["Wl"], bp["bl"].reshape(1, H * C), 2000)
        xr = _mm_bias(h, bp["Wr"], bp["br"].reshape(1, H * C), 2000)
        g1, g2 = _sc_gather([xl, xr], [src, dst])
        wrows_t = _gat_edge(edge_features, g1, g2, m_w, cb,
                            _att_matrix(bp["att"]))
        acc3 = _sc_scatter_add(wrows_t.reshape(NS, CPT, E), dst)
        acc = (acc3.reshape(NC, NS, NPAD, CPT)
               .transpose(0, 2, 1, 3).reshape(NC, NPAD, ACCW)[:, :N])
        h = _finalize(acc[0], acc[1],
                      bp["bias"].reshape(1, H * C),
                      bp["ln_g"].reshape(1, HID), bp["ln_b"].reshape(1, HID))

    nh = params["node_head"]
    eh = params["edge_head"]
    em = params["edge_mlp"]
    rc = params["recon"]
    w6p = jnp.pad(eh["W"], ((0, 0), (0, 16 - EDGE_CLS)))
    b6p = jnp.pad(eh["b"], (0, 16 - EDGE_CLS))
    zpad = jnp.zeros((HID, 112), jnp.float32)
    w_all = jnp.concatenate(
        [nh["W"], rc["W"], em["W1"][HID:], em["W1"][:HID], w6p, zpad],
        axis=1)
    b_all = jnp.concatenate(
        [nh["b"], rc["b"], jnp.zeros((HID,), jnp.float32), em["b1"], b6p,
         jnp.zeros((112,), jnp.float32)]).reshape(1, -1)
    node_preds, recon, v, uz = _node_head(h, w_all, b_all)

    guz, gv = _sc_gather([uz, v], [src, dst])
    edge_preds, exist = _head_edge(guz, gv, em["W2"].reshape(1, HID),
                                   em["b2"].reshape(1, 1))

    return (node_preds, edge_preds, exist, recon)
